# baseline (device time: 36595 ns/iter reference)
import jax
import jax.numpy as jnp
from jax import lax
from jax.experimental import pallas as pl
from jax.experimental.pallas import tpu as pltpu

M = 2048
N = 1024
HALF_M = M // 2
HALF_N = N // 2
K = 16
CH = HALF_M // K


def kernel(x):
    def body(x_hbm, out_hbm, xl_ref, recv_ref, s_ref,
             sem_local, sem_out, sem_sx, sem_rx, sem_sy, sem_ry):
        a = lax.axis_index("x")
        b = lax.axis_index("y")

        barrier_sem = pltpu.get_barrier_semaphore()
        pl.semaphore_signal(barrier_sem, inc=1, device_id=(1 - a, b),
                            device_id_type=pl.DeviceIdType.MESH)
        pl.semaphore_signal(barrier_sem, inc=1, device_id=(a, 1 - b),
                            device_id_type=pl.DeviceIdType.MESH)
        pl.semaphore_wait(barrier_sem, 2)

        base = b * HALF_M

        local_cp = pltpu.make_async_copy(
            x_hbm.at[0, pl.ds(base, HALF_M), pl.ds(a * HALF_N, HALF_N)],
            xl_ref,
            sem_local,
        )
        local_cp.start()

        rdma_x = []
        for k in range(K):
            rows_k = pl.ds(base + k * CH, CH)
            r = pltpu.make_async_remote_copy(
                src_ref=x_hbm.at[0, rows_k, pl.ds((1 - a) * HALF_N, HALF_N)],
                dst_ref=recv_ref.at[k],
                send_sem=sem_sx.at[k],
                recv_sem=sem_rx.at[k],
                device_id=(1 - a, b),
                device_id_type=pl.DeviceIdType.MESH,
            )
            r.start()
            rdma_x.append(r)

        local_cp.wait()

        rdma_y = []
        out_cp = []
        for k in range(K):
            rows_k = pl.ds(base + k * CH, CH)
            rdma_x[k].wait_recv()
            s_ref[k] = xl_ref[k * CH:(k + 1) * CH, :] + recv_ref[k]
            c = pltpu.make_async_copy(
                s_ref.at[k], out_hbm.at[rows_k, :], sem_out.at[k]
            )
            c.start()
            out_cp.append(c)
            r = pltpu.make_async_remote_copy(
                src_ref=s_ref.at[k],
                dst_ref=out_hbm.at[rows_k, :],
                send_sem=sem_sy.at[k],
                recv_sem=sem_ry.at[k],
                device_id=(a, 1 - b),
                device_id_type=pl.DeviceIdType.MESH,
            )
            r.start()
            rdma_y.append(r)

        for k in range(K):
            rdma_x[k].wait_send()
            out_cp[k].wait()
            rdma_y[k].wait()

    return pl.pallas_call(
        body,
        out_shape=jax.ShapeDtypeStruct((M, HALF_N), jnp.float32),
        in_specs=[pl.BlockSpec(memory_space=pl.ANY)],
        out_specs=pl.BlockSpec(memory_space=pl.ANY),
        scratch_shapes=[
            pltpu.VMEM((HALF_M, HALF_N), jnp.float32),
            pltpu.VMEM((K, CH, HALF_N), jnp.float32),
            pltpu.VMEM((K, CH, HALF_N), jnp.float32),
            pltpu.SemaphoreType.DMA,
            pltpu.SemaphoreType.DMA((K,)),
            pltpu.SemaphoreType.DMA((K,)),
            pltpu.SemaphoreType.DMA((K,)),
            pltpu.SemaphoreType.DMA((K,)),
            pltpu.SemaphoreType.DMA((K,)),
        ],
        compiler_params=pltpu.CompilerParams(collective_id=0),
    )(x)
